# baseline (speedup 1.0000x reference)
import functools

import jax
import jax.numpy as jnp
from jax import lax
from jax.experimental import pallas as pl
from jax.experimental.pallas import tpu as pltpu
from jax.experimental.pallas import tpu_sc as plsc

N = 10000
NP = 10240
E = 320000
D = 128
NC = 2
NS = 16
NW = NC * NS
CH = 128
GW = 80
GC = 40
EPG = NW * GW
EP = EPG * CH
ROWS_PER_SUB = NP // NS
BR = 1024
GRID = NP // BR

_mesh = plsc.VectorSubcoreMesh(core_axis_name="c", subcore_axis_name="s",
                               num_cores=NC, num_subcores=NS)


@functools.partial(
    pl.kernel,
    out_type=jax.ShapeDtypeStruct((NC, NP), jnp.float32),
    mesh=_mesh,
    scratch_types=[
        pltpu.VMEM((GW, CH), jnp.int32),
        pltpu.VMEM((CH,), jnp.float32),
        pltpu.VMEM((ROWS_PER_SUB,), jnp.float32),
        pltpu.VMEM_SHARED((NP,), jnp.float32),
    ],
)
def _deg_kernel(dst_hbm, degp_hbm, idx_all, ones_v, zeros_v, deg_sh):
    c = lax.axis_index("c")
    s = lax.axis_index("s")
    w = s * NC + c
    for i in range(CH // 16):
        ones_v[pl.ds(i * 16, 16)] = jnp.ones((16,), jnp.float32)
    for i in range(ROWS_PER_SUB // 16):
        zeros_v[pl.ds(i * 16, 16)] = jnp.zeros((16,), jnp.float32)
    pltpu.sync_copy(dst_hbm.at[pl.ds(w * GW, GW)], idx_all)
    pltpu.sync_copy(zeros_v, deg_sh.at[pl.ds(s * ROWS_PER_SUB, ROWS_PER_SUB)])
    plsc.subcore_barrier()

    def step(g, carry):
        pltpu.sync_copy(ones_v, deg_sh.at[idx_all.at[g]], add=True)
        return carry

    lax.fori_loop(0, GW, step, 0)
    plsc.subcore_barrier()
    pltpu.sync_copy(deg_sh.at[pl.ds(s * ROWS_PER_SUB, ROWS_PER_SUB)],
                    degp_hbm.at[c, pl.ds(s * ROWS_PER_SUB, ROWS_PER_SUB)])


@functools.partial(
    pl.kernel,
    out_type=jax.ShapeDtypeStruct((NC, NP, D), jnp.float32),
    mesh=_mesh,
    scratch_types=[
        pltpu.VMEM((GC, CH), jnp.int32),
        pltpu.VMEM((GC, CH), jnp.int32),
        pltpu.VMEM((CH, D), jnp.float32),
        pltpu.VMEM((CH, D), jnp.float32),
        pltpu.VMEM_SHARED((NP, D), jnp.float32),
        pltpu.SemaphoreType.DMA,
        pltpu.SemaphoreType.DMA,
    ],
)
def _msg_kernel(hs_hbm, src_hbm, dst_hbm, out_hbm,
                isrc_all, idst_all, rows_a, rows_b, acc_sh, sem_a, sem_b):
    c = lax.axis_index("c")
    s = lax.axis_index("s")
    w = s * NC + c
    rows = (rows_a, rows_b)
    sems = (sem_a, sem_b)

    pltpu.sync_copy(hs_hbm.at[pl.ds(s * ROWS_PER_SUB, ROWS_PER_SUB)],
                    acc_sh.at[pl.ds(s * ROWS_PER_SUB, ROWS_PER_SUB)])
    plsc.subcore_barrier()

    for chunk in range(GW // GC):
        base = w * GW + chunk * GC
        pltpu.sync_copy(src_hbm.at[pl.ds(base, GC)], isrc_all)
        pltpu.sync_copy(dst_hbm.at[pl.ds(base, GC)], idst_all)

        def step(g, carry):
            pltpu.async_copy(hs_hbm.at[isrc_all.at[g]], rows_a, sem_a).wait()
            pltpu.sync_copy(rows_a, acc_sh.at[idst_all.at[g]], add=True)
            return carry

        lax.fori_loop(0, GC, step, 0)

    plsc.subcore_barrier()
    pltpu.sync_copy(acc_sh.at[pl.ds(s * ROWS_PER_SUB, ROWS_PER_SUB)],
                    out_hbm.at[c, pl.ds(s * ROWS_PER_SUB, ROWS_PER_SUB)])


def _b1_body(x_ref, w_ref, d0_ref, d1_ref, hs_ref, dinv_ref):
    deg = d0_ref[...] + d1_ref[...] + 1.0
    dinv = lax.rsqrt(deg)
    dinv_ref[...] = dinv
    h = jnp.dot(x_ref[...], w_ref[...], preferred_element_type=jnp.float32)
    hs_ref[...] = h * dinv


def _b1(xp, W1, d0, d1):
    return pl.pallas_call(
        _b1_body,
        grid=(GRID,),
        in_specs=[
            pl.BlockSpec((BR, D), lambda i: (i, 0)),
            pl.BlockSpec((D, D), lambda i: (0, 0)),
            pl.BlockSpec((BR, 1), lambda i: (i, 0)),
            pl.BlockSpec((BR, 1), lambda i: (i, 0)),
        ],
        out_specs=[
            pl.BlockSpec((BR, D), lambda i: (i, 0)),
            pl.BlockSpec((BR, 1), lambda i: (i, 0)),
        ],
        out_shape=[
            jax.ShapeDtypeStruct((NP, D), jnp.float32),
            jax.ShapeDtypeStruct((NP, 1), jnp.float32),
        ],
    )(xp, W1, d0, d1)


def _b2_body(p0_ref, p1_ref, hs_ref, dinv_ref, b_ref, w_ref, out_ref):
    dinv = dinv_ref[...]
    pre = dinv * (p0_ref[...] + p1_ref[...] - hs_ref[...]) + b_ref[...]
    h = jnp.maximum(pre, 0.0)
    out_ref[...] = jnp.dot(h, w_ref[...],
                           preferred_element_type=jnp.float32) * dinv


def _b2(p0, p1, hs, dinv, b, W2):
    return pl.pallas_call(
        _b2_body,
        grid=(GRID,),
        in_specs=[
            pl.BlockSpec((BR, D), lambda i: (i, 0)),
            pl.BlockSpec((BR, D), lambda i: (i, 0)),
            pl.BlockSpec((BR, D), lambda i: (i, 0)),
            pl.BlockSpec((BR, 1), lambda i: (i, 0)),
            pl.BlockSpec((1, D), lambda i: (0, 0)),
            pl.BlockSpec((D, D), lambda i: (0, 0)),
        ],
        out_specs=pl.BlockSpec((BR, D), lambda i: (i, 0)),
        out_shape=jax.ShapeDtypeStruct((NP, D), jnp.float32),
    )(p0, p1, hs, dinv, b, W2)


def _b3_body(p0_ref, p1_ref, hs_ref, dinv_ref, b_ref, w_ref, fb_ref, out_ref):
    pre = dinv_ref[...] * (p0_ref[...] + p1_ref[...] - hs_ref[...]) + b_ref[...]
    h = jnp.maximum(pre, 0.0)
    out_ref[...] = jnp.dot(h, w_ref[...],
                           preferred_element_type=jnp.float32) + fb_ref[...]


def _b3(p0, p1, hs, dinv, b, fcW, fcb):
    return pl.pallas_call(
        _b3_body,
        grid=(GRID,),
        in_specs=[
            pl.BlockSpec((BR, D), lambda i: (i, 0)),
            pl.BlockSpec((BR, D), lambda i: (i, 0)),
            pl.BlockSpec((BR, D), lambda i: (i, 0)),
            pl.BlockSpec((BR, 1), lambda i: (i, 0)),
            pl.BlockSpec((1, D), lambda i: (0, 0)),
            pl.BlockSpec((D, D), lambda i: (0, 0)),
            pl.BlockSpec((1, D), lambda i: (0, 0)),
        ],
        out_specs=pl.BlockSpec((BR, D), lambda i: (i, 0)),
        out_shape=jax.ShapeDtypeStruct((NP, D), jnp.float32),
    )(p0, p1, hs, dinv, b, fcW, fcb)


def kernel(x, edge_index, batch, W1, b1, W2, b2, fc_W, fc_b):
    del batch
    src = jnp.pad(edge_index[0], (0, EP - E), constant_values=N)
    dst = jnp.pad(edge_index[1], (0, EP - E), constant_values=N)
    src2 = src.reshape(EPG, CH)
    dst2 = dst.reshape(EPG, CH)
    xp = jnp.pad(x.astype(jnp.float32), ((0, NP - N), (0, 0)))

    degp = _deg_kernel(dst2)
    d0 = degp[0][:, None]
    d1 = degp[1][:, None]

    hs1, dinv = _b1(xp, W1, d0, d1)
    p = _msg_kernel(hs1, src2, dst2)
    hs2 = _b2(p[0], p[1], hs1, dinv, b1[None, :], W2)
    q = _msg_kernel(hs2, src2, dst2)
    out = _b3(q[0], q[1], hs2, dinv, b2[None, :], fc_W, fc_b[None, :])
    return out[:N]


# speedup vs baseline: 1.0013x; 1.0013x over previous
import functools

import jax
import jax.numpy as jnp
from jax import lax
from jax.experimental import pallas as pl
from jax.experimental.pallas import tpu as pltpu
from jax.experimental.pallas import tpu_sc as plsc

N = 10000
NP = 10240
E = 320000
D = 128
NC = 2
NS = 16
NW = NC * NS
CH = 128
GW = 80
EPG = NW * GW
EP = EPG * CH
ROWS_PER_SUB = NP // NS
BR = 1024
GRID = NP // BR

_mesh = plsc.VectorSubcoreMesh(core_axis_name="c", subcore_axis_name="s",
                               num_cores=NC, num_subcores=NS)


@functools.partial(
    pl.kernel,
    out_type=jax.ShapeDtypeStruct((NC, NP), jnp.float32),
    mesh=_mesh,
    scratch_types=[
        pltpu.VMEM((GW, CH), jnp.int32),
        pltpu.VMEM((CH,), jnp.float32),
        pltpu.VMEM((ROWS_PER_SUB,), jnp.float32),
        pltpu.VMEM_SHARED((NP,), jnp.float32),
    ],
)
def _deg_kernel(dst_hbm, degp_hbm, idx_all, ones_v, zeros_v, deg_sh):
    c = lax.axis_index("c")
    s = lax.axis_index("s")
    w = s * NC + c
    for i in range(CH // 16):
        ones_v[pl.ds(i * 16, 16)] = jnp.ones((16,), jnp.float32)
    for i in range(ROWS_PER_SUB // 16):
        zeros_v[pl.ds(i * 16, 16)] = jnp.zeros((16,), jnp.float32)
    pltpu.sync_copy(dst_hbm.at[pl.ds(w * GW, GW)], idx_all)
    pltpu.sync_copy(zeros_v, deg_sh.at[pl.ds(s * ROWS_PER_SUB, ROWS_PER_SUB)])
    plsc.subcore_barrier()

    def step(g, carry):
        pltpu.sync_copy(ones_v, deg_sh.at[idx_all.at[g]], add=True)
        return carry

    lax.fori_loop(0, GW, step, 0)
    plsc.subcore_barrier()
    pltpu.sync_copy(deg_sh.at[pl.ds(s * ROWS_PER_SUB, ROWS_PER_SUB)],
                    degp_hbm.at[c, pl.ds(s * ROWS_PER_SUB, ROWS_PER_SUB)])


@functools.partial(
    pl.kernel,
    out_type=jax.ShapeDtypeStruct((NC, NP, D), jnp.float32),
    mesh=_mesh,
    scratch_types=[
        pltpu.VMEM((GW, CH), jnp.int32),
        pltpu.VMEM((GW, CH), jnp.int32),
        pltpu.VMEM((CH, D), jnp.float32),
        pltpu.VMEM_SHARED((NP, D), jnp.float32),
    ],
)
def _msg_kernel(hs_hbm, src_hbm, dst_hbm, out_hbm,
                isrc_all, idst_all, rows, acc_sh):
    c = lax.axis_index("c")
    s = lax.axis_index("s")
    w = s * NC + c

    pltpu.sync_copy(hs_hbm.at[pl.ds(s * ROWS_PER_SUB, ROWS_PER_SUB)],
                    acc_sh.at[pl.ds(s * ROWS_PER_SUB, ROWS_PER_SUB)])
    plsc.subcore_barrier()

    pltpu.sync_copy(src_hbm.at[pl.ds(w * GW, GW)], isrc_all)
    pltpu.sync_copy(dst_hbm.at[pl.ds(w * GW, GW)], idst_all)

    def step(g, carry):
        pltpu.sync_copy(hs_hbm.at[isrc_all.at[g]], rows)
        pltpu.sync_copy(rows, acc_sh.at[idst_all.at[g]], add=True)
        return carry

    lax.fori_loop(0, GW, step, 0)

    plsc.subcore_barrier()
    pltpu.sync_copy(acc_sh.at[pl.ds(s * ROWS_PER_SUB, ROWS_PER_SUB)],
                    out_hbm.at[c, pl.ds(s * ROWS_PER_SUB, ROWS_PER_SUB)])


def _b1_body(x_ref, w_ref, d0_ref, d1_ref, hs_ref, dinv_ref):
    deg = d0_ref[...] + d1_ref[...] + 1.0
    dinv = lax.rsqrt(deg)
    dinv_ref[...] = dinv
    h = jnp.dot(x_ref[...], w_ref[...], preferred_element_type=jnp.float32)
    hs_ref[...] = h * dinv


def _b1(xp, W1, d0, d1):
    return pl.pallas_call(
        _b1_body,
        grid=(GRID,),
        in_specs=[
            pl.BlockSpec((BR, D), lambda i: (i, 0)),
            pl.BlockSpec((D, D), lambda i: (0, 0)),
            pl.BlockSpec((BR, 1), lambda i: (i, 0)),
            pl.BlockSpec((BR, 1), lambda i: (i, 0)),
        ],
        out_specs=[
            pl.BlockSpec((BR, D), lambda i: (i, 0)),
            pl.BlockSpec((BR, 1), lambda i: (i, 0)),
        ],
        out_shape=[
            jax.ShapeDtypeStruct((NP, D), jnp.float32),
            jax.ShapeDtypeStruct((NP, 1), jnp.float32),
        ],
    )(xp, W1, d0, d1)


def _b2_body(p0_ref, p1_ref, hs_ref, dinv_ref, b_ref, w_ref, out_ref):
    dinv = dinv_ref[...]
    pre = dinv * (p0_ref[...] + p1_ref[...] - hs_ref[...]) + b_ref[...]
    h = jnp.maximum(pre, 0.0)
    out_ref[...] = jnp.dot(h, w_ref[...],
                           preferred_element_type=jnp.float32) * dinv


def _b2(p0, p1, hs, dinv, b, W2):
    return pl.pallas_call(
        _b2_body,
        grid=(GRID,),
        in_specs=[
            pl.BlockSpec((BR, D), lambda i: (i, 0)),
            pl.BlockSpec((BR, D), lambda i: (i, 0)),
            pl.BlockSpec((BR, D), lambda i: (i, 0)),
            pl.BlockSpec((BR, 1), lambda i: (i, 0)),
            pl.BlockSpec((1, D), lambda i: (0, 0)),
            pl.BlockSpec((D, D), lambda i: (0, 0)),
        ],
        out_specs=pl.BlockSpec((BR, D), lambda i: (i, 0)),
        out_shape=jax.ShapeDtypeStruct((NP, D), jnp.float32),
    )(p0, p1, hs, dinv, b, W2)


def _b3_body(p0_ref, p1_ref, hs_ref, dinv_ref, b_ref, w_ref, fb_ref, out_ref):
    pre = dinv_ref[...] * (p0_ref[...] + p1_ref[...] - hs_ref[...]) + b_ref[...]
    h = jnp.maximum(pre, 0.0)
    out_ref[...] = jnp.dot(h, w_ref[...],
                           preferred_element_type=jnp.float32) + fb_ref[...]


def _b3(p0, p1, hs, dinv, b, fcW, fcb):
    return pl.pallas_call(
        _b3_body,
        grid=(GRID,),
        in_specs=[
            pl.BlockSpec((BR, D), lambda i: (i, 0)),
            pl.BlockSpec((BR, D), lambda i: (i, 0)),
            pl.BlockSpec((BR, D), lambda i: (i, 0)),
            pl.BlockSpec((BR, 1), lambda i: (i, 0)),
            pl.BlockSpec((1, D), lambda i: (0, 0)),
            pl.BlockSpec((D, D), lambda i: (0, 0)),
            pl.BlockSpec((1, D), lambda i: (0, 0)),
        ],
        out_specs=pl.BlockSpec((BR, D), lambda i: (i, 0)),
        out_shape=jax.ShapeDtypeStruct((NP, D), jnp.float32),
    )(p0, p1, hs, dinv, b, fcW, fcb)


def kernel(x, edge_index, batch, W1, b1, W2, b2, fc_W, fc_b):
    del batch
    src = jnp.pad(edge_index[0], (0, EP - E), constant_values=N)
    dst = jnp.pad(edge_index[1], (0, EP - E), constant_values=N)
    src2 = src.reshape(EPG, CH)
    dst2 = dst.reshape(EPG, CH)
    xp = jnp.pad(x.astype(jnp.float32), ((0, NP - N), (0, 0)))

    degp = _deg_kernel(dst2)
    d0 = degp[0][:, None]
    d1 = degp[1][:, None]

    hs1, dinv = _b1(xp, W1, d0, d1)
    p = _msg_kernel(hs1, src2, dst2)
    hs2 = _b2(p[0], p[1], hs1, dinv, b1[None, :], W2)
    q = _msg_kernel(hs2, src2, dst2)
    out = _b3(q[0], q[1], hs2, dinv, b2[None, :], fc_W, fc_b[None, :])
    return out[:N]


# speedup vs baseline: 1.1003x; 1.0989x over previous
import functools

import jax
import jax.numpy as jnp
from jax import lax
from jax.experimental import pallas as pl
from jax.experimental.pallas import tpu as pltpu
from jax.experimental.pallas import tpu_sc as plsc

N = 10000
NP = 10240
E = 320000
D = 128
NC = 2
NS = 16
NW = NC * NS
CH = 128
GW = 80
HG = 40
EPG = NW * GW
EP = EPG * CH
ROWS_PER_SUB = NP // NS
BR = 1024
GRID = NP // BR

_mesh = plsc.VectorSubcoreMesh(core_axis_name="c", subcore_axis_name="s",
                               num_cores=NC, num_subcores=NS)


@functools.partial(
    pl.kernel,
    out_type=jax.ShapeDtypeStruct((NC, NP), jnp.float32),
    mesh=_mesh,
    scratch_types=[
        pltpu.VMEM((GW, CH), jnp.int32),
        pltpu.VMEM((CH,), jnp.float32),
        pltpu.VMEM((ROWS_PER_SUB,), jnp.float32),
        pltpu.VMEM_SHARED((NP,), jnp.float32),
    ],
)
def _deg_kernel(dst_hbm, degp_hbm, idx_all, ones_v, zeros_v, deg_sh):
    c = lax.axis_index("c")
    s = lax.axis_index("s")
    w = s * NC + c
    for i in range(CH // 16):
        ones_v[pl.ds(i * 16, 16)] = jnp.ones((16,), jnp.float32)
    for i in range(ROWS_PER_SUB // 16):
        zeros_v[pl.ds(i * 16, 16)] = jnp.zeros((16,), jnp.float32)
    pltpu.sync_copy(dst_hbm.at[pl.ds(w * GW, GW)], idx_all)
    pltpu.sync_copy(zeros_v, deg_sh.at[pl.ds(s * ROWS_PER_SUB, ROWS_PER_SUB)])
    plsc.subcore_barrier()

    def step(g, carry):
        pltpu.sync_copy(ones_v, deg_sh.at[idx_all.at[g]], add=True)
        return carry

    lax.fori_loop(0, GW, step, 0)
    plsc.subcore_barrier()
    pltpu.sync_copy(deg_sh.at[pl.ds(s * ROWS_PER_SUB, ROWS_PER_SUB)],
                    degp_hbm.at[c, pl.ds(s * ROWS_PER_SUB, ROWS_PER_SUB)])


@functools.partial(
    pl.kernel,
    out_type=jax.ShapeDtypeStruct((NC, NP, D), jnp.float32),
    mesh=_mesh,
    scratch_types=[
        pltpu.VMEM((HG, CH), jnp.int32),
        pltpu.VMEM((HG, CH), jnp.int32),
        pltpu.VMEM((CH, D), jnp.float32),
        pltpu.VMEM((CH, D), jnp.float32),
        pltpu.VMEM_SHARED((NP, D), jnp.float32),
        pltpu.SemaphoreType.DMA,
        pltpu.SemaphoreType.DMA,
    ],
)
def _msg_kernel(hs_hbm, src_hbm, dst_hbm, out_hbm,
                isrc_c, idst_c, r0, r1, acc_sh, sem0, sem1):
    c = lax.axis_index("c")
    s = lax.axis_index("s")
    w = s * NC + c
    rows = (r0, r1)
    sems = (sem0, sem1)

    pltpu.sync_copy(hs_hbm.at[pl.ds(s * ROWS_PER_SUB, ROWS_PER_SUB)],
                    acc_sh.at[pl.ds(s * ROWS_PER_SUB, ROWS_PER_SUB)])
    plsc.subcore_barrier()

    for h in range(GW // HG):
        base = w * GW + h * HG
        pltpu.sync_copy(src_hbm.at[pl.ds(base, HG)], isrc_c)
        pltpu.sync_copy(dst_hbm.at[pl.ds(base, HG)], idst_c)

        for b in range(2):
            pltpu.async_copy(hs_hbm.at[isrc_c.at[b]], rows[b], sems[b])

        def steady(i, carry):
            g0 = i * 2
            for b in range(2):
                pltpu.make_async_copy(hs_hbm.at[pl.ds(0, CH)], rows[b],
                                      sems[b]).wait()
                pltpu.sync_copy(rows[b], acc_sh.at[idst_c.at[g0 + b]],
                                add=True)
                pltpu.async_copy(hs_hbm.at[isrc_c.at[g0 + b + 2]],
                                 rows[b], sems[b])
            return carry

        lax.fori_loop(0, HG // 2 - 1, steady, 0)

        for b in range(2):
            g = HG - 2 + b
            pltpu.make_async_copy(hs_hbm.at[pl.ds(0, CH)], rows[b],
                                  sems[b]).wait()
            pltpu.sync_copy(rows[b], acc_sh.at[idst_c.at[g]], add=True)

    plsc.subcore_barrier()
    pltpu.sync_copy(acc_sh.at[pl.ds(s * ROWS_PER_SUB, ROWS_PER_SUB)],
                    out_hbm.at[c, pl.ds(s * ROWS_PER_SUB, ROWS_PER_SUB)])


def _b1_body(x_ref, w_ref, d0_ref, d1_ref, hs_ref, dinv_ref):
    deg = d0_ref[...] + d1_ref[...] + 1.0
    dinv = lax.rsqrt(deg)
    dinv_ref[...] = dinv
    h = jnp.dot(x_ref[...], w_ref[...], preferred_element_type=jnp.float32)
    hs_ref[...] = h * dinv


def _b1(xp, W1, d0, d1):
    return pl.pallas_call(
        _b1_body,
        grid=(GRID,),
        in_specs=[
            pl.BlockSpec((BR, D), lambda i: (i, 0)),
            pl.BlockSpec((D, D), lambda i: (0, 0)),
            pl.BlockSpec((BR, 1), lambda i: (i, 0)),
            pl.BlockSpec((BR, 1), lambda i: (i, 0)),
        ],
        out_specs=[
            pl.BlockSpec((BR, D), lambda i: (i, 0)),
            pl.BlockSpec((BR, 1), lambda i: (i, 0)),
        ],
        out_shape=[
            jax.ShapeDtypeStruct((NP, D), jnp.float32),
            jax.ShapeDtypeStruct((NP, 1), jnp.float32),
        ],
    )(xp, W1, d0, d1)


def _b2_body(p0_ref, p1_ref, hs_ref, dinv_ref, b_ref, w_ref, out_ref):
    dinv = dinv_ref[...]
    pre = dinv * (p0_ref[...] + p1_ref[...] - hs_ref[...]) + b_ref[...]
    h = jnp.maximum(pre, 0.0)
    out_ref[...] = jnp.dot(h, w_ref[...],
                           preferred_element_type=jnp.float32) * dinv


def _b2(p0, p1, hs, dinv, b, W2):
    return pl.pallas_call(
        _b2_body,
        grid=(GRID,),
        in_specs=[
            pl.BlockSpec((BR, D), lambda i: (i, 0)),
            pl.BlockSpec((BR, D), lambda i: (i, 0)),
            pl.BlockSpec((BR, D), lambda i: (i, 0)),
            pl.BlockSpec((BR, 1), lambda i: (i, 0)),
            pl.BlockSpec((1, D), lambda i: (0, 0)),
            pl.BlockSpec((D, D), lambda i: (0, 0)),
        ],
        out_specs=pl.BlockSpec((BR, D), lambda i: (i, 0)),
        out_shape=jax.ShapeDtypeStruct((NP, D), jnp.float32),
    )(p0, p1, hs, dinv, b, W2)


def _b3_body(p0_ref, p1_ref, hs_ref, dinv_ref, b_ref, w_ref, fb_ref, out_ref):
    pre = dinv_ref[...] * (p0_ref[...] + p1_ref[...] - hs_ref[...]) + b_ref[...]
    h = jnp.maximum(pre, 0.0)
    out_ref[...] = jnp.dot(h, w_ref[...],
                           preferred_element_type=jnp.float32) + fb_ref[...]


def _b3(p0, p1, hs, dinv, b, fcW, fcb):
    return pl.pallas_call(
        _b3_body,
        grid=(GRID,),
        in_specs=[
            pl.BlockSpec((BR, D), lambda i: (i, 0)),
            pl.BlockSpec((BR, D), lambda i: (i, 0)),
            pl.BlockSpec((BR, D), lambda i: (i, 0)),
            pl.BlockSpec((BR, 1), lambda i: (i, 0)),
            pl.BlockSpec((1, D), lambda i: (0, 0)),
            pl.BlockSpec((D, D), lambda i: (0, 0)),
            pl.BlockSpec((1, D), lambda i: (0, 0)),
        ],
        out_specs=pl.BlockSpec((BR, D), lambda i: (i, 0)),
        out_shape=jax.ShapeDtypeStruct((NP, D), jnp.float32),
    )(p0, p1, hs, dinv, b, fcW, fcb)


def kernel(x, edge_index, batch, W1, b1, W2, b2, fc_W, fc_b):
    del batch
    src = jnp.pad(edge_index[0], (0, EP - E), constant_values=N)
    dst = jnp.pad(edge_index[1], (0, EP - E), constant_values=N)
    src2 = src.reshape(EPG, CH)
    dst2 = dst.reshape(EPG, CH)
    xp = jnp.pad(x.astype(jnp.float32), ((0, NP - N), (0, 0)))

    degp = _deg_kernel(dst2)
    d0 = degp[0][:, None]
    d1 = degp[1][:, None]

    hs1, dinv = _b1(xp, W1, d0, d1)
    p = _msg_kernel(hs1, src2, dst2)
    hs2 = _b2(p[0], p[1], hs1, dinv, b1[None, :], W2)
    q = _msg_kernel(hs2, src2, dst2)
    out = _b3(q[0], q[1], hs2, dinv, b2[None, :], fc_W, fc_b[None, :])
    return out[:N]
